# baseline (speedup 1.0000x reference)
import functools
import math

import numpy as np
import jax
import jax.numpy as jnp
from jax import lax
from jax.experimental import pallas as pl
from jax.experimental.pallas import tpu as pltpu
from jax.experimental.pallas import tpu_sc as plsc


def _d2xy(n, d):
    rx = ry = 0
    x = y = 0
    t = d
    s = 1
    while s < n:
        rx = 1 & (t // 2)
        ry = 1 & (t ^ rx)
        if ry == 0:
            if rx == 1:
                x = s - 1 - x
                y = s - 1 - y
            x, y = y, x
        x += s * rx
        y += s * ry
        t //= 4
        s *= 2
    return x, y


def _hilbert_perm(seq_len):
    grid_size = int(math.ceil(math.sqrt(seq_len)))
    g = 1
    while g < grid_size:
        g *= 2
    n_levels = int(math.log2(g))
    gg = 2 ** n_levels
    hilbert_indices = []
    for d in range(gg * gg):
        x, y = _d2xy(gg, d)
        hilbert_indices.append(y * gg + x)
    valid = [idx for idx in hilbert_indices if idx < seq_len]
    if len(valid) < seq_len:
        remaining = sorted(set(range(seq_len)) - set(valid))
        valid.extend(remaining)
    return np.array(valid[:seq_len], dtype=np.int32)


def _make_sc_gather(R, D, per_w, C):
    n_chunks = per_w // C
    assert n_chunks % 2 == 0 and n_chunks >= 4
    mesh = plsc.VectorSubcoreMesh(core_axis_name="c", subcore_axis_name="s")
    info = plsc.get_sparse_core_info()
    NC = info.num_cores

    @functools.partial(
        pl.kernel,
        mesh=mesh,
        out_type=jax.ShapeDtypeStruct((R, D), jnp.float32),
        scratch_types=[
            pltpu.VMEM((per_w,), jnp.int32),
            pltpu.VMEM((C, D), jnp.float32),
            pltpu.VMEM((C, D), jnp.float32),
            pltpu.SemaphoreType.DMA,
            pltpu.SemaphoreType.DMA,
            pltpu.SemaphoreType.DMA,
            pltpu.SemaphoreType.DMA,
        ],
    )
    def k(table_hbm, idx_hbm, out_hbm, idx_v, buf0, buf1, g0, g1, w0, w1):
        wid = lax.axis_index("s") * NC + lax.axis_index("c")
        base = wid * per_w
        pltpu.sync_copy(idx_hbm.at[pl.ds(base, per_w)], idx_v)

        bufs = (buf0, buf1)
        gsems = (g0, g1)
        wsems = (w0, w1)

        def gstart(ci, b):
            pltpu.async_copy(
                table_hbm.at[idx_v.at[pl.ds(ci * C, C)]], bufs[b], gsems[b])

        def gwait(ci, b):
            pltpu.make_async_copy(
                table_hbm.at[idx_v.at[pl.ds(ci * C, C)]], bufs[b],
                gsems[b]).wait()

        def wstart(ci, b):
            pltpu.async_copy(
                bufs[b], out_hbm.at[pl.ds(base + ci * C, C)], wsems[b])

        def wwait(ci, b):
            pltpu.make_async_copy(
                bufs[b], out_hbm.at[pl.ds(base + ci * C, C)],
                wsems[b]).wait()

        gstart(0, 0)
        gstart(1, 1)

        def body(i, _):
            ci = i * 2
            gwait(ci, 0)
            wstart(ci, 0)
            gwait(ci + 1, 1)
            wstart(ci + 1, 1)

            @pl.when(i < n_chunks // 2 - 1)
            def _():
                wwait(ci, 0)
                gstart(ci + 2, 0)
                wwait(ci + 1, 1)
                gstart(ci + 3, 1)

            return 0

        lax.fori_loop(0, n_chunks // 2, body, 0)
        wwait(n_chunks - 2, 0)
        wwait(n_chunks - 1, 1)

    return k


def kernel(tensor):
    B, S, D = tensor.shape
    R = B * S
    perm = _hilbert_perm(S)
    gidx = (np.arange(B, dtype=np.int32)[:, None] * S + perm[None, :]).reshape(-1)
    gidx = jnp.asarray(gidx)

    info = plsc.get_sparse_core_info()
    NW = info.num_cores * info.num_subcores
    per_w = R // NW
    C = 8

    table = tensor.reshape(R, D)
    out = _make_sc_gather(R, D, per_w, C)(table, gidx)
    return out.reshape(B, S, D)


# speedup vs baseline: 1.0568x; 1.0568x over previous
import functools
import math

import numpy as np
import jax
import jax.numpy as jnp
from jax import lax
from jax.experimental import pallas as pl
from jax.experimental.pallas import tpu as pltpu
from jax.experimental.pallas import tpu_sc as plsc


def _d2xy(n, d):
    rx = ry = 0
    x = y = 0
    t = d
    s = 1
    while s < n:
        rx = 1 & (t // 2)
        ry = 1 & (t ^ rx)
        if ry == 0:
            if rx == 1:
                x = s - 1 - x
                y = s - 1 - y
            x, y = y, x
        x += s * rx
        y += s * ry
        t //= 4
        s *= 2
    return x, y


def _hilbert_perm(seq_len):
    grid_size = int(math.ceil(math.sqrt(seq_len)))
    g = 1
    while g < grid_size:
        g *= 2
    n_levels = int(math.log2(g))
    gg = 2 ** n_levels
    hilbert_indices = []
    for d in range(gg * gg):
        x, y = _d2xy(gg, d)
        hilbert_indices.append(y * gg + x)
    valid = [idx for idx in hilbert_indices if idx < seq_len]
    if len(valid) < seq_len:
        remaining = sorted(set(range(seq_len)) - set(valid))
        valid.extend(remaining)
    return np.array(valid[:seq_len], dtype=np.int32)


def _make_sc_gather(R, D, per_w, C):
    n_chunks = per_w // C
    assert n_chunks % 2 == 0 and n_chunks >= 4
    mesh = plsc.VectorSubcoreMesh(core_axis_name="c", subcore_axis_name="s")
    info = plsc.get_sparse_core_info()
    NC = info.num_cores

    @functools.partial(
        pl.kernel,
        mesh=mesh,
        out_type=jax.ShapeDtypeStruct((R, D), jnp.float32),
        scratch_types=[
            pltpu.VMEM((per_w,), jnp.int32),
            pltpu.VMEM((C, D), jnp.float32),
            pltpu.VMEM((C, D), jnp.float32),
            pltpu.SemaphoreType.DMA,
            pltpu.SemaphoreType.DMA,
            pltpu.SemaphoreType.DMA,
            pltpu.SemaphoreType.DMA,
        ],
    )
    def k(table_hbm, idx_hbm, out_hbm, idx_v, buf0, buf1, g0, g1, w0, w1):
        wid = lax.axis_index("s") * NC + lax.axis_index("c")
        base = wid * per_w
        pltpu.sync_copy(idx_hbm.at[pl.ds(base, per_w)], idx_v)

        bufs = (buf0, buf1)
        gsems = (g0, g1)
        wsems = (w0, w1)

        def gstart(ci, b):
            pltpu.async_copy(
                table_hbm.at[idx_v.at[pl.ds(ci * C, C)]], bufs[b], gsems[b])

        def gwait(ci, b):
            pltpu.make_async_copy(
                table_hbm.at[idx_v.at[pl.ds(ci * C, C)]], bufs[b],
                gsems[b]).wait()

        def wstart(ci, b):
            pltpu.async_copy(
                bufs[b], out_hbm.at[pl.ds(base + ci * C, C)], wsems[b])

        def wwait(ci, b):
            pltpu.make_async_copy(
                bufs[b], out_hbm.at[pl.ds(base + ci * C, C)],
                wsems[b]).wait()

        gstart(0, 0)
        gstart(1, 1)

        def body(i, _):
            ci = i * 2
            gwait(ci, 0)
            wstart(ci, 0)
            gwait(ci + 1, 1)
            wstart(ci + 1, 1)

            @pl.when(i < n_chunks // 2 - 1)
            def _():
                wwait(ci, 0)
                gstart(ci + 2, 0)
                wwait(ci + 1, 1)
                gstart(ci + 3, 1)

            return 0

        lax.fori_loop(0, n_chunks // 2, body, 0)
        wwait(n_chunks - 2, 0)
        wwait(n_chunks - 1, 1)

    return k


def kernel(tensor):
    B, S, D = tensor.shape
    R = B * S
    perm = _hilbert_perm(S)
    gidx = (np.arange(B, dtype=np.int32)[:, None] * S + perm[None, :]).reshape(-1)
    gidx = jnp.asarray(gidx)

    info = plsc.get_sparse_core_info()
    NW = info.num_cores * info.num_subcores
    per_w = R // NW
    C = 16

    table = tensor.reshape(R, D)
    out = _make_sc_gather(R, D, per_w, C)(table, gidx)
    return out.reshape(B, S, D)


# speedup vs baseline: 1.1170x; 1.0570x over previous
import functools
import math

import numpy as np
import jax
import jax.numpy as jnp
from jax import lax
from jax.experimental import pallas as pl
from jax.experimental.pallas import tpu as pltpu
from jax.experimental.pallas import tpu_sc as plsc


def _d2xy(n, d):
    rx = ry = 0
    x = y = 0
    t = d
    s = 1
    while s < n:
        rx = 1 & (t // 2)
        ry = 1 & (t ^ rx)
        if ry == 0:
            if rx == 1:
                x = s - 1 - x
                y = s - 1 - y
            x, y = y, x
        x += s * rx
        y += s * ry
        t //= 4
        s *= 2
    return x, y


def _hilbert_perm(seq_len):
    grid_size = int(math.ceil(math.sqrt(seq_len)))
    g = 1
    while g < grid_size:
        g *= 2
    n_levels = int(math.log2(g))
    gg = 2 ** n_levels
    hilbert_indices = []
    for d in range(gg * gg):
        x, y = _d2xy(gg, d)
        hilbert_indices.append(y * gg + x)
    valid = [idx for idx in hilbert_indices if idx < seq_len]
    if len(valid) < seq_len:
        remaining = sorted(set(range(seq_len)) - set(valid))
        valid.extend(remaining)
    return np.array(valid[:seq_len], dtype=np.int32)


def _make_sc_gather(R, D, per_w):
    CA = 16
    CB = 8
    SUP = CA + 2 * CB
    n_super = per_w // SUP
    assert per_w % SUP == 0 and n_super >= 2
    mesh = plsc.VectorSubcoreMesh(core_axis_name="c", subcore_axis_name="s")
    info = plsc.get_sparse_core_info()
    NC = info.num_cores

    @functools.partial(
        pl.kernel,
        mesh=mesh,
        out_type=jax.ShapeDtypeStruct((R, D), jnp.float32),
        scratch_types=[
            pltpu.VMEM((per_w,), jnp.int32),
            pltpu.VMEM((CA, D), jnp.float32),
            pltpu.VMEM((CA, D), jnp.float32),
            pltpu.VMEM_SHARED((16, 2, CB, D), jnp.float32),
            pltpu.SemaphoreType.DMA,
            pltpu.SemaphoreType.DMA,
            pltpu.SemaphoreType.DMA,
            pltpu.SemaphoreType.DMA,
            pltpu.SemaphoreType.DMA,
            pltpu.SemaphoreType.DMA,
            pltpu.SemaphoreType.DMA,
            pltpu.SemaphoreType.DMA,
        ],
    )
    def k(table_hbm, idx_hbm, out_hbm, idx_v, buf0, buf1, shbuf,
          ga0, ga1, wa0, wa1, gb0, gb1, wb0, wb1):
        wid = lax.axis_index("s") * NC + lax.axis_index("c")
        sid = lax.axis_index("s")
        base = wid * per_w
        pltpu.sync_copy(idx_hbm.at[pl.ds(base, per_w)], idx_v)

        abufs = (buf0, buf1)
        gasems = (ga0, ga1)
        wasems = (wa0, wa1)
        gbsems = (gb0, gb1)
        wbsems = (wb0, wb1)

        def astart(i, b):
            pltpu.async_copy(
                table_hbm.at[idx_v.at[pl.ds(i * SUP, CA)]], abufs[b],
                gasems[b])

        def await_(i, b):
            pltpu.make_async_copy(
                table_hbm.at[idx_v.at[pl.ds(i * SUP, CA)]], abufs[b],
                gasems[b]).wait()

        def awstart(i, b):
            pltpu.async_copy(
                abufs[b], out_hbm.at[pl.ds(base + i * SUP, CA)], wasems[b])

        def awwait(i, b):
            pltpu.make_async_copy(
                abufs[b], out_hbm.at[pl.ds(base + i * SUP, CA)],
                wasems[b]).wait()

        def bstart(i, j):
            ivec = idx_v[pl.ds(i * SUP + CA, 2 * CB)]
            for kk in range(CB):
                src = ivec[j * CB + kk]
                pltpu.async_copy(
                    table_hbm.at[pl.ds(src, 1)],
                    shbuf.at[sid, j, pl.ds(kk, 1)], gbsems[j])

        def bwait(i, j):
            pltpu.make_async_copy(
                table_hbm.at[pl.ds(0, CB)], shbuf.at[sid, j],
                gbsems[j]).wait()

        def bwstart(i, j):
            pltpu.async_copy(
                shbuf.at[sid, j],
                out_hbm.at[pl.ds(base + i * SUP + CA + j * CB, CB)],
                wbsems[j])

        def bwwait(i, j):
            pltpu.make_async_copy(
                shbuf.at[sid, j],
                out_hbm.at[pl.ds(base + i * SUP + CA + j * CB, CB)],
                wbsems[j]).wait()

        n_pairs = n_super // 2
        assert n_super % 2 == 0

        astart(0, 0)
        bstart(0, 0)
        bstart(0, 1)
        astart(1, 1)

        def body(ii, _):
            i0 = ii * 2
            i1 = i0 + 1

            await_(i0, 0)
            awstart(i0, 0)
            bwait(i0, 0)
            bwstart(i0, 0)
            bwait(i0, 1)
            bwstart(i0, 1)
            bwwait(i0, 0)
            bstart(i1, 0)
            bwwait(i0, 1)
            bstart(i1, 1)

            @pl.when(ii < n_pairs - 1)
            def _():
                awwait(i0, 0)
                astart(i0 + 2, 0)

            await_(i1, 1)
            awstart(i1, 1)
            bwait(i1, 0)
            bwstart(i1, 0)
            bwait(i1, 1)
            bwstart(i1, 1)

            @pl.when(ii < n_pairs - 1)
            def _():
                bwwait(i1, 0)
                bstart(i1 + 1, 0)
                bwwait(i1, 1)
                bstart(i1 + 1, 1)
                awwait(i1, 1)
                astart(i1 + 2, 1)

            return 0

        lax.fori_loop(0, n_pairs, body, 0)
        awwait(n_super - 2, 0)
        awwait(n_super - 1, 1)
        bwwait(n_super - 1, 0)
        bwwait(n_super - 1, 1)

    return k


def kernel(tensor):
    B, S, D = tensor.shape
    R = B * S
    perm = _hilbert_perm(S)
    gidx = (np.arange(B, dtype=np.int32)[:, None] * S + perm[None, :]).reshape(-1)
    gidx = jnp.asarray(gidx)

    info = plsc.get_sparse_core_info()
    NW = info.num_cores * info.num_subcores
    per_w = R // NW

    table = tensor.reshape(R, D)
    out = _make_sc_gather(R, D, per_w)(table, gidx)
    return out.reshape(B, S, D)


# speedup vs baseline: 1.1178x; 1.0007x over previous
import functools
import math

import numpy as np
import jax
import jax.numpy as jnp
from jax import lax
from jax.experimental import pallas as pl
from jax.experimental.pallas import tpu as pltpu
from jax.experimental.pallas import tpu_sc as plsc


def _d2xy(n, d):
    rx = ry = 0
    x = y = 0
    t = d
    s = 1
    while s < n:
        rx = 1 & (t // 2)
        ry = 1 & (t ^ rx)
        if ry == 0:
            if rx == 1:
                x = s - 1 - x
                y = s - 1 - y
            x, y = y, x
        x += s * rx
        y += s * ry
        t //= 4
        s *= 2
    return x, y


def _hilbert_perm(seq_len):
    grid_size = int(math.ceil(math.sqrt(seq_len)))
    g = 1
    while g < grid_size:
        g *= 2
    n_levels = int(math.log2(g))
    gg = 2 ** n_levels
    hilbert_indices = []
    for d in range(gg * gg):
        x, y = _d2xy(gg, d)
        hilbert_indices.append(y * gg + x)
    valid = [idx for idx in hilbert_indices if idx < seq_len]
    if len(valid) < seq_len:
        remaining = sorted(set(range(seq_len)) - set(valid))
        valid.extend(remaining)
    return np.array(valid[:seq_len], dtype=np.int32)


def _make_sc_gather(R, D, per_w0, per_w1):
    CA = 16
    CB = 8
    SUP = CA + 2 * CB
    n_super0 = per_w0 // SUP
    n_super1 = per_w1 // SUP
    per_w_max = max(per_w0, per_w1)
    for pw, ns in ((per_w0, n_super0), (per_w1, n_super1)):
        assert pw % SUP == 0 and ns >= 2 and ns % 2 == 0
    mesh = plsc.VectorSubcoreMesh(core_axis_name="c", subcore_axis_name="s")
    info = plsc.get_sparse_core_info()
    NC = info.num_cores

    @functools.partial(
        pl.kernel,
        mesh=mesh,
        out_type=jax.ShapeDtypeStruct((R, D), jnp.float32),
        scratch_types=[
            pltpu.VMEM((per_w_max,), jnp.int32),
            pltpu.VMEM((CA, D), jnp.float32),
            pltpu.VMEM((CA, D), jnp.float32),
            pltpu.VMEM_SHARED((16, 2, CB, D), jnp.float32),
            pltpu.SemaphoreType.DMA,
            pltpu.SemaphoreType.DMA,
            pltpu.SemaphoreType.DMA,
            pltpu.SemaphoreType.DMA,
            pltpu.SemaphoreType.DMA,
            pltpu.SemaphoreType.DMA,
            pltpu.SemaphoreType.DMA,
            pltpu.SemaphoreType.DMA,
        ],
    )
    def k(table_hbm, idx_hbm, out_hbm, idx_v, buf0, buf1, shbuf,
          ga0, ga1, wa0, wa1, gb0, gb1, wb0, wb1):
        cid = lax.axis_index("c")
        sid = lax.axis_index("s")
        on_core0 = cid == 0
        base = jnp.where(on_core0, sid * per_w0,
                         16 * per_w0 + sid * per_w1)
        n_super = jnp.where(on_core0, n_super0, n_super1)
        n_pairs = jnp.where(on_core0, n_super0 // 2, n_super1 // 2)
        pltpu.sync_copy(idx_hbm.at[pl.ds(base, per_w_max)], idx_v)

        abufs = (buf0, buf1)
        gasems = (ga0, ga1)
        wasems = (wa0, wa1)
        gbsems = (gb0, gb1)
        wbsems = (wb0, wb1)

        def astart(i, b):
            pltpu.async_copy(
                table_hbm.at[idx_v.at[pl.ds(i * SUP, CA)]], abufs[b],
                gasems[b])

        def await_(i, b):
            pltpu.make_async_copy(
                table_hbm.at[idx_v.at[pl.ds(i * SUP, CA)]], abufs[b],
                gasems[b]).wait()

        def awstart(i, b):
            pltpu.async_copy(
                abufs[b], out_hbm.at[pl.ds(base + i * SUP, CA)], wasems[b])

        def awwait(i, b):
            pltpu.make_async_copy(
                abufs[b], out_hbm.at[pl.ds(base + i * SUP, CA)],
                wasems[b]).wait()

        def bstart(i, j):
            ivec = idx_v[pl.ds(i * SUP + CA, 2 * CB)]
            for kk in range(CB):
                src = ivec[j * CB + kk]
                pltpu.async_copy(
                    table_hbm.at[pl.ds(src, 1)],
                    shbuf.at[sid, j, pl.ds(kk, 1)], gbsems[j])

        def bwait(i, j):
            pltpu.make_async_copy(
                table_hbm.at[pl.ds(0, CB)], shbuf.at[sid, j],
                gbsems[j]).wait()

        def bwstart(i, j):
            pltpu.async_copy(
                shbuf.at[sid, j],
                out_hbm.at[pl.ds(base + i * SUP + CA + j * CB, CB)],
                wbsems[j])

        def bwwait(i, j):
            pltpu.make_async_copy(
                shbuf.at[sid, j],
                out_hbm.at[pl.ds(base + i * SUP + CA + j * CB, CB)],
                wbsems[j]).wait()

        astart(0, 0)
        bstart(0, 0)
        bstart(0, 1)
        astart(1, 1)

        def body(ii, _):
            i0 = ii * 2
            i1 = i0 + 1

            await_(i0, 0)
            awstart(i0, 0)
            bwait(i0, 0)
            bwstart(i0, 0)
            bwait(i0, 1)
            bwstart(i0, 1)
            bwwait(i0, 0)
            bstart(i1, 0)
            bwwait(i0, 1)
            bstart(i1, 1)

            @pl.when(ii < n_pairs - 1)
            def _():
                awwait(i0, 0)
                astart(i0 + 2, 0)

            await_(i1, 1)
            awstart(i1, 1)
            bwait(i1, 0)
            bwstart(i1, 0)
            bwait(i1, 1)
            bwstart(i1, 1)

            @pl.when(ii < n_pairs - 1)
            def _():
                bwwait(i1, 0)
                bstart(i1 + 1, 0)
                bwwait(i1, 1)
                bstart(i1 + 1, 1)
                awwait(i1, 1)
                astart(i1 + 2, 1)

            return 0

        lax.fori_loop(0, n_pairs, body, 0)
        awwait(n_super - 2, 0)
        awwait(n_super - 1, 1)
        bwwait(n_super - 1, 0)
        bwwait(n_super - 1, 1)

    return k


def kernel(tensor):
    B, S, D = tensor.shape
    R = B * S
    perm = _hilbert_perm(S)
    gidx = (np.arange(B, dtype=np.int32)[:, None] * S + perm[None, :]).reshape(-1)
    gidx = jnp.asarray(gidx)

    info = plsc.get_sparse_core_info()
    NW = info.num_cores * info.num_subcores
    per_w = R // NW
    per_w0 = per_w + 64
    per_w1 = per_w - 64
    pad = 16 * per_w0 + 15 * per_w1 + max(per_w0, per_w1) - R
    gidx = jnp.concatenate([gidx, jnp.zeros((pad,), jnp.int32)])

    table = tensor.reshape(R, D)
    out = _make_sc_gather(R, D, per_w0, per_w1)(table, gidx)
    return out.reshape(B, S, D)


# speedup vs baseline: 1.1217x; 1.0034x over previous
import functools
import math

import numpy as np
import jax
import jax.numpy as jnp
from jax import lax
from jax.experimental import pallas as pl
from jax.experimental.pallas import tpu as pltpu
from jax.experimental.pallas import tpu_sc as plsc


def _d2xy(n, d):
    rx = ry = 0
    x = y = 0
    t = d
    s = 1
    while s < n:
        rx = 1 & (t // 2)
        ry = 1 & (t ^ rx)
        if ry == 0:
            if rx == 1:
                x = s - 1 - x
                y = s - 1 - y
            x, y = y, x
        x += s * rx
        y += s * ry
        t //= 4
        s *= 2
    return x, y


def _hilbert_perm(seq_len):
    grid_size = int(math.ceil(math.sqrt(seq_len)))
    g = 1
    while g < grid_size:
        g *= 2
    n_levels = int(math.log2(g))
    gg = 2 ** n_levels
    hilbert_indices = []
    for d in range(gg * gg):
        x, y = _d2xy(gg, d)
        hilbert_indices.append(y * gg + x)
    valid = [idx for idx in hilbert_indices if idx < seq_len]
    if len(valid) < seq_len:
        remaining = sorted(set(range(seq_len)) - set(valid))
        valid.extend(remaining)
    return np.array(valid[:seq_len], dtype=np.int32)


def _make_sc_gather(R, D, per_w0, per_w1):
    CA = 16
    CB = 8
    SUP = CA + 2 * CB
    n_super0 = per_w0 // SUP
    n_super1 = per_w1 // SUP
    per_w_max = max(per_w0, per_w1)
    for pw, ns in ((per_w0, n_super0), (per_w1, n_super1)):
        assert pw % SUP == 0 and ns >= 2 and ns % 2 == 0
    mesh = plsc.VectorSubcoreMesh(core_axis_name="c", subcore_axis_name="s")
    info = plsc.get_sparse_core_info()
    NC = info.num_cores

    @functools.partial(
        pl.kernel,
        mesh=mesh,
        out_type=jax.ShapeDtypeStruct((R, D), jnp.float32),
        scratch_types=[
            pltpu.VMEM((per_w_max,), jnp.int32),
            pltpu.VMEM((CA, D), jnp.float32),
            pltpu.VMEM((CA, D), jnp.float32),
            pltpu.VMEM_SHARED((16, 2, CB, D), jnp.float32),
            pltpu.SemaphoreType.DMA,
            pltpu.SemaphoreType.DMA,
            pltpu.SemaphoreType.DMA,
            pltpu.SemaphoreType.DMA,
            pltpu.SemaphoreType.DMA,
            pltpu.SemaphoreType.DMA,
            pltpu.SemaphoreType.DMA,
            pltpu.SemaphoreType.DMA,
        ],
    )
    def k(table_hbm, idx_hbm, out_hbm, idx_v, buf0, buf1, shbuf,
          ga0, ga1, wa0, wa1, gb0, gb1, wb0, wb1):
        cid = lax.axis_index("c")
        sid = lax.axis_index("s")
        on_core0 = cid == 0
        base = jnp.where(on_core0, sid * per_w0,
                         16 * per_w0 + sid * per_w1)
        n_super = jnp.where(on_core0, n_super0, n_super1)
        n_pairs = jnp.where(on_core0, n_super0 // 2, n_super1 // 2)
        pltpu.sync_copy(idx_hbm.at[pl.ds(base, per_w_max)], idx_v)

        abufs = (buf0, buf1)
        gasems = (ga0, ga1)
        wasems = (wa0, wa1)
        gbsems = (gb0, gb1)
        wbsems = (wb0, wb1)

        def astart(i, b):
            pltpu.async_copy(
                table_hbm.at[idx_v.at[pl.ds(i * SUP, CA)]], abufs[b],
                gasems[b])

        def await_(i, b):
            pltpu.make_async_copy(
                table_hbm.at[idx_v.at[pl.ds(i * SUP, CA)]], abufs[b],
                gasems[b]).wait()

        def awstart(i, b):
            pltpu.async_copy(
                abufs[b], out_hbm.at[pl.ds(base + i * SUP, CA)], wasems[b])

        def awwait(i, b):
            pltpu.make_async_copy(
                abufs[b], out_hbm.at[pl.ds(base + i * SUP, CA)],
                wasems[b]).wait()

        def bstart(i, j):
            ivec = idx_v[pl.ds(i * SUP + CA, 2 * CB)]
            for kk in range(CB):
                src = ivec[j * CB + kk]
                pltpu.async_copy(
                    table_hbm.at[pl.ds(src, 1)],
                    shbuf.at[sid, j, pl.ds(kk, 1)], gbsems[j])

        def bwait(i, j):
            pltpu.make_async_copy(
                table_hbm.at[pl.ds(0, CB)], shbuf.at[sid, j],
                gbsems[j]).wait()

        def bwstart(i, j):
            pltpu.async_copy(
                shbuf.at[sid, j],
                out_hbm.at[pl.ds(base + i * SUP + CA + j * CB, CB)],
                wbsems[j])

        def bwwait(i, j):
            pltpu.make_async_copy(
                shbuf.at[sid, j],
                out_hbm.at[pl.ds(base + i * SUP + CA + j * CB, CB)],
                wbsems[j]).wait()

        astart(0, 0)
        bstart(0, 0)
        bstart(0, 1)
        astart(1, 1)

        def body(ii, _):
            i0 = ii * 2
            i1 = i0 + 1

            await_(i0, 0)
            awstart(i0, 0)
            bwait(i0, 0)
            bwstart(i0, 0)
            bwait(i0, 1)
            bwstart(i0, 1)
            bwwait(i0, 0)
            bstart(i1, 0)
            bwwait(i0, 1)
            bstart(i1, 1)

            @pl.when(ii < n_pairs - 1)
            def _():
                awwait(i0, 0)
                astart(i0 + 2, 0)

            await_(i1, 1)
            awstart(i1, 1)
            bwait(i1, 0)
            bwstart(i1, 0)
            bwait(i1, 1)
            bwstart(i1, 1)

            @pl.when(ii < n_pairs - 1)
            def _():
                bwwait(i1, 0)
                bstart(i1 + 1, 0)
                bwwait(i1, 1)
                bstart(i1 + 1, 1)
                awwait(i1, 1)
                astart(i1 + 2, 1)

            return 0

        lax.fori_loop(0, n_pairs, body, 0)
        awwait(n_super - 2, 0)
        awwait(n_super - 1, 1)
        bwwait(n_super - 1, 0)
        bwwait(n_super - 1, 1)

    return k


def kernel(tensor):
    B, S, D = tensor.shape
    R = B * S
    perm = _hilbert_perm(S)
    gidx = (np.arange(B, dtype=np.int32)[:, None] * S + perm[None, :]).reshape(-1)
    gidx = jnp.asarray(gidx)

    info = plsc.get_sparse_core_info()
    NW = info.num_cores * info.num_subcores
    per_w = R // NW
    per_w0 = per_w - 64
    per_w1 = per_w + 64
    pad = 16 * per_w0 + 15 * per_w1 + max(per_w0, per_w1) - R
    gidx = jnp.concatenate([gidx, jnp.zeros((pad,), jnp.int32)])

    table = tensor.reshape(R, D)
    out = _make_sc_gather(R, D, per_w0, per_w1)(table, gidx)
    return out.reshape(B, S, D)


# speedup vs baseline: 1.1389x; 1.0153x over previous
import functools
import math

import numpy as np
import jax
import jax.numpy as jnp
from jax import lax
from jax.experimental import pallas as pl
from jax.experimental.pallas import tpu as pltpu
from jax.experimental.pallas import tpu_sc as plsc


def _d2xy(n, d):
    rx = ry = 0
    x = y = 0
    t = d
    s = 1
    while s < n:
        rx = 1 & (t // 2)
        ry = 1 & (t ^ rx)
        if ry == 0:
            if rx == 1:
                x = s - 1 - x
                y = s - 1 - y
            x, y = y, x
        x += s * rx
        y += s * ry
        t //= 4
        s *= 2
    return x, y


def _hilbert_perm(seq_len):
    grid_size = int(math.ceil(math.sqrt(seq_len)))
    g = 1
    while g < grid_size:
        g *= 2
    n_levels = int(math.log2(g))
    gg = 2 ** n_levels
    hilbert_indices = []
    for d in range(gg * gg):
        x, y = _d2xy(gg, d)
        hilbert_indices.append(y * gg + x)
    valid = [idx for idx in hilbert_indices if idx < seq_len]
    if len(valid) < seq_len:
        remaining = sorted(set(range(seq_len)) - set(valid))
        valid.extend(remaining)
    return np.array(valid[:seq_len], dtype=np.int32)


def _make_sc_gather(R, D, per_w):
    CA = 16
    CB = 8
    SUP = CA + 2 * CB
    n_super = per_w // SUP
    assert per_w % SUP == 0 and n_super >= 4 and n_super % 2 == 0
    mesh = plsc.VectorSubcoreMesh(core_axis_name="c", subcore_axis_name="s")
    info = plsc.get_sparse_core_info()
    NC = info.num_cores

    @functools.partial(
        pl.kernel,
        mesh=mesh,
        out_type=jax.ShapeDtypeStruct((R, D), jnp.float32),
        scratch_types=[
            pltpu.VMEM((per_w,), jnp.int32),
            pltpu.VMEM((CA, D), jnp.float32),
            pltpu.VMEM((CA, D), jnp.float32),
            pltpu.VMEM_SHARED((16, 2, CB, D), jnp.float32),
            pltpu.SemaphoreType.DMA,
            pltpu.SemaphoreType.DMA,
            pltpu.SemaphoreType.DMA,
            pltpu.SemaphoreType.DMA,
            pltpu.SemaphoreType.DMA,
            pltpu.SemaphoreType.DMA,
            pltpu.SemaphoreType.DMA,
            pltpu.SemaphoreType.DMA,
        ],
    )
    def k(table_hbm, idx_hbm, out_hbm, idx_v, buf0, buf1, shbuf,
          ga0, ga1, wa0, wa1, gb0, gb1, wb0, wb1):
        cid = lax.axis_index("c")
        sid = lax.axis_index("s")
        base = (sid * NC + cid) * per_w
        pltpu.sync_copy(idx_hbm.at[pl.ds(base, per_w)], idx_v)

        abufs = (buf0, buf1)
        gasems = (ga0, ga1)
        wasems = (wa0, wa1)
        gbsems = (gb0, gb1)
        wbsems = (wb0, wb1)


        def astart(i, b):
            pltpu.async_copy(
                table_hbm.at[idx_v.at[pl.ds(i * SUP, CA)]], abufs[b],
                gasems[b])

        def await_(i, b):
            pltpu.make_async_copy(
                table_hbm.at[idx_v.at[pl.ds(i * SUP, CA)]], abufs[b],
                gasems[b]).wait()

        def awstart(i, b):
            pltpu.async_copy(
                abufs[b], out_hbm.at[pl.ds(base + i * SUP, CA)], wasems[b])

        def awwait(i, b):
            pltpu.make_async_copy(
                abufs[b], out_hbm.at[pl.ds(base + i * SUP, CA)],
                wasems[b]).wait()


        def bstart(i, j):
            ivec = idx_v[pl.ds(i * SUP + CA, 2 * CB)]
            for kk in range(CB):
                src = ivec[j * CB + kk]
                pltpu.async_copy(
                    table_hbm.at[pl.ds(src, 1)],
                    shbuf.at[sid, j, pl.ds(kk, 1)], gbsems[j])

        def bwait(i, j):
            pltpu.make_async_copy(
                table_hbm.at[pl.ds(0, CB)], shbuf.at[sid, j],
                gbsems[j]).wait()

        def bwstart(i, j):
            pltpu.async_copy(
                shbuf.at[sid, j],
                out_hbm.at[pl.ds(base + i * SUP + CA + j * CB, CB)],
                wbsems[j])

        def bwwait(i, j):
            pltpu.make_async_copy(
                shbuf.at[sid, j],
                out_hbm.at[pl.ds(base + i * SUP + CA + j * CB, CB)],
                wbsems[j]).wait()

        n_pairs = n_super // 2

        astart(0, 0)
        bstart(0, 0)
        bstart(0, 1)
        astart(1, 1)

        def body(ii, _):
            i0 = ii * 2
            i1 = i0 + 1

            await_(i0, 0)
            awstart(i0, 0)
            bwait(i0, 0)
            bwstart(i0, 0)
            bwait(i0, 1)
            bwstart(i0, 1)
            bwwait(i0, 0)
            bstart(i1, 0)
            bwwait(i0, 1)
            bstart(i1, 1)

            @pl.when(ii < n_pairs - 1)
            def _():
                awwait(i0, 0)
                astart(i0 + 2, 0)

            await_(i1, 1)
            awstart(i1, 1)
            bwait(i1, 0)
            bwstart(i1, 0)
            bwait(i1, 1)
            bwstart(i1, 1)

            @pl.when(ii < n_pairs - 1)
            def _():
                bwwait(i1, 0)
                bstart(i1 + 1, 0)
                bwwait(i1, 1)
                bstart(i1 + 1, 1)
                awwait(i1, 1)
                astart(i1 + 2, 1)

            return 0

        lax.fori_loop(0, n_pairs, body, 0)
        awwait(n_super - 2, 0)
        awwait(n_super - 1, 1)
        bwwait(n_super - 1, 0)
        bwwait(n_super - 1, 1)

    return k


def kernel(tensor):
    B, S, D = tensor.shape
    R = B * S
    perm = _hilbert_perm(S)
    gidx = (np.arange(B, dtype=np.int32)[:, None] * S + perm[None, :]).reshape(-1)
    gidx = jnp.asarray(gidx)

    info = plsc.get_sparse_core_info()
    NW = info.num_cores * info.num_subcores
    per_w = R // NW

    table = tensor.reshape(R, D)
    out = _make_sc_gather(R, D, per_w)(table, gidx)
    return out.reshape(B, S, D)
